# baseline (speedup 1.0000x reference)
import functools

import jax
import jax.numpy as jnp
from jax import lax
from jax.experimental import pallas as pl
from jax.experimental.pallas import tpu as pltpu
from jax.experimental.pallas import tpu_sc as plsc

UNK = 1000
D = 64
SUB = 128
C = 1024
NSUB = C // SUB
RINGB = 16
RING = RINGB * SUB


def _make_kernel(L, NW, per_w):
    nch = per_w // C
    mesh = plsc.VectorSubcoreMesh(core_axis_name="c", subcore_axis_name="s")

    @functools.partial(
        pl.kernel,
        mesh=mesh,
        compiler_params=pltpu.CompilerParams(use_tc_tiling_on_sc=False,
                                             needs_layout_passes=False),
        out_type=jax.ShapeDtypeStruct((L + 8, D), jnp.float32),
        scratch_types=[
            pltpu.VMEM((C,), jnp.int32),
            pltpu.VMEM((C, D), jnp.float32),
            pltpu.VMEM((RINGB, SUB), jnp.int32),
            pltpu.VMEM((RINGB, SUB), jnp.int32),
            pltpu.VMEM((SUB, D), jnp.float32),
            pltpu.SemaphoreType.DMA,
            pltpu.SemaphoreType.DMA,
        ],
    )
    def body(ids_hbm, glove_hbm, unk_hbm, out_hbm,
             idx_v, rows_v, uid_v, upos_v, ubuf_v, sem_g, sem_u):
        wid = lax.axis_index("s") * 2 + lax.axis_index("c")
        base = pl.multiple_of(wid * per_w, C)
        lane = lax.iota(jnp.int32, 16)

        def fire_block(b, carry):
            r = lax.rem(b, RINGB)
            pltpu.async_copy(unk_hbm.at[uid_v.at[r]], ubuf_v, sem_u).wait()
            pltpu.async_copy(ubuf_v, out_hbm.at[upos_v.at[r]], sem_u).wait()
            return carry

        def chunk(g, carry):
            ucur, fdone = carry
            b0 = pl.multiple_of(base + g * C, C)
            pltpu.sync_copy(ids_hbm.at[pl.ds(b0, C)], idx_v)
            for k in range(C // 16):
                o = k * 16
                ids = idx_v[pl.ds(o, 16)]
                m = ids < jnp.full((16,), UNK, jnp.int32)
                mi = m.astype(jnp.int32)
                idx_v[pl.ds(o, 16)] = jnp.where(
                    m, jnp.zeros((16,), jnp.int32),
                    ids - jnp.full((16,), UNK, jnp.int32))
                excl = plsc.cumsum(mi) - mi
                tgt = lax.rem(jnp.full((16,), ucur, jnp.int32) + excl,
                              jnp.full((16,), RING, jnp.int32))
                row = lax.shift_right_logical(tgt, jnp.full((16,), 7, jnp.int32))
                col = lax.bitwise_and(tgt, jnp.full((16,), SUB - 1, jnp.int32))
                pos = jnp.full((16,), b0 + o, jnp.int32) + lane
                plsc.store_scatter(uid_v, [row, col], ids, mask=m)
                plsc.store_scatter(upos_v, [row, col], pos, mask=m)
                ucur = ucur + jnp.sum(mi)
            cps = [
                pltpu.async_copy(
                    glove_hbm.at[idx_v.at[pl.ds(j * SUB, SUB)]],
                    rows_v.at[pl.ds(j * SUB, SUB)],
                    sem_g,
                )
                for j in range(NSUB)
            ]
            for cp in cps:
                cp.wait()
            pltpu.sync_copy(rows_v, out_hbm.at[pl.ds(b0, C)])
            nf = lax.shift_right_logical(ucur, 7)
            fdone = lax.fori_loop(lax.shift_right_logical(fdone, 7), nf,
                                  fire_block, fdone)
            fdone = lax.shift_left(nf, 7)
            return ucur, fdone

        ucur, fdone = lax.fori_loop(0, nch, chunk, (0, 0))

        @pl.when(ucur > fdone)
        def _flush():
            r = lax.rem(lax.shift_right_logical(fdone, 7), RINGB)
            rem = lax.bitwise_and(ucur, SUB - 1)
            for k in range(SUB // 16):
                col = jnp.full((16,), k * 16, jnp.int32) + lane
                m = col >= jnp.full((16,), rem, jnp.int32)
                plsc.store_scatter(uid_v, [jnp.full((16,), r, jnp.int32), col],
                                   jnp.zeros((16,), jnp.int32), mask=m)
                plsc.store_scatter(upos_v, [jnp.full((16,), r, jnp.int32), col],
                                   jnp.full((16,), L, jnp.int32), mask=m)
            fire_block(lax.shift_right_logical(fdone, 7), 0)

    return body


def kernel(context, glove_table, unk_table):
    b, t = context.shape
    L = b * t
    NW = 32
    per_w = L // NW
    assert per_w % C == 0
    flat = context.reshape(L)
    out = _make_kernel(L, NW, per_w)(flat, glove_table, unk_table)
    return out[:L].reshape(b, t, D)


# speedup vs baseline: 1.2877x; 1.2877x over previous
import functools

import jax
import jax.numpy as jnp
from jax import lax
from jax.experimental import pallas as pl
from jax.experimental.pallas import tpu as pltpu
from jax.experimental.pallas import tpu_sc as plsc

UNK = 1000
D = 64
SUB = 128
C = 512
NSUB = C // SUB


def _make_kernel(L, NW, per_w):
    nch = per_w // C
    assert nch % 2 == 0 and nch >= 4
    pcap = per_w + 16
    mesh = plsc.VectorSubcoreMesh(core_axis_name="c", subcore_axis_name="s")

    @functools.partial(
        pl.kernel,
        mesh=mesh,
        compiler_params=pltpu.CompilerParams(use_tc_tiling_on_sc=False,
                                             needs_layout_passes=False),
        out_type=jax.ShapeDtypeStruct((L, D), jnp.float32),
        scratch_types=[
            pltpu.VMEM((C,), jnp.int32),
            pltpu.VMEM((C,), jnp.int32),
            pltpu.VMEM((C, D), jnp.float32),
            pltpu.VMEM((C, D), jnp.float32),
            pltpu.VMEM((pcap,), jnp.int32),
            pltpu.VMEM((pcap,), jnp.int32),
            pltpu.VMEM((1, SUB), jnp.int32),
            pltpu.VMEM((SUB, D), jnp.float32),
            pltpu.SemaphoreType.DMA,
            pltpu.SemaphoreType.DMA,
            pltpu.SemaphoreType.DMA,
            pltpu.SemaphoreType.DMA,
            pltpu.SemaphoreType.DMA,
        ],
    )
    def body(ids_hbm, glove_hbm, unk_hbm, out_hbm,
             idx_a, idx_b, rows_a, rows_b, uid_v, upos_v, pos2d_v, ubuf_v,
             sem_ga, sem_gb, sem_oa, sem_ob, sem_u):
        wid = lax.axis_index("s") * 2 + lax.axis_index("c")
        base = pl.multiple_of(wid * per_w, C)
        lane = lax.iota(jnp.int32, 16)
        idxs = [idx_a, idx_b]
        rowss = [rows_a, rows_b]
        sem_g = [sem_ga, sem_gb]
        sem_o = [sem_oa, sem_ob]

        def compute(g, idx_v, cur):
            b0 = pl.multiple_of(base + g * C, C)
            pltpu.sync_copy(ids_hbm.at[pl.ds(b0, C)], idx_v)
            for k in range(C // 16):
                o = k * 16
                ids = idx_v[pl.ds(o, 16)]
                m = ids < jnp.full((16,), UNK, jnp.int32)
                mi = jnp.where(m, jnp.full((16,), 1, jnp.int32),
                               jnp.zeros((16,), jnp.int32))
                idx_v[pl.ds(o, 16)] = jnp.where(
                    m, jnp.zeros((16,), jnp.int32),
                    ids - jnp.full((16,), UNK, jnp.int32))
                pos = jnp.full((16,), b0 + o, jnp.int32) + lane
                plsc.store_compressed(uid_v.at[pl.ds(cur, 16)], ids, mask=m)
                plsc.store_compressed(upos_v.at[pl.ds(cur, 16)], pos, mask=m)
                cur = cur + jnp.sum(mi)
            return cur

        def fire_gathers(g, p):
            idx_v = idxs[p]
            rows_v = rowss[p]
            for j in range(NSUB):
                pltpu.async_copy(
                    glove_hbm.at[idx_v.at[pl.ds(j * SUB, SUB)]],
                    rows_v.at[pl.ds(j * SUB, SUB)],
                    sem_g[p],
                )

        def wait_gathers(p):
            pltpu.make_async_copy(glove_hbm.at[pl.ds(0, C)], rowss[p],
                                  sem_g[p]).wait()

        def fire_copyout(g, p):
            b0 = pl.multiple_of(base + g * C, C)
            pltpu.async_copy(rowss[p], out_hbm.at[pl.ds(b0, C)], sem_o[p])

        def wait_copyout(p):
            pltpu.make_async_copy(rowss[p], out_hbm.at[pl.ds(0, C)],
                                  sem_o[p]).wait()

        def step(g, p, cur, wait_prev_gather, wait_prev_copyout):
            if wait_prev_gather:
                wait_gathers(1 - p)
                fire_copyout(g - 1, 1 - p)
            if wait_prev_copyout:
                wait_copyout(p)
            cur = compute(g, idxs[p], cur)
            fire_gathers(g, p)
            return cur

        cur = step(0, 0, 0, False, False)
        cur = step(1, 1, cur, True, False)

        def pair(i, cur):
            g = i * 2
            cur = step(g, 0, cur, True, True)
            cur = step(g + 1, 1, cur, True, True)
            return cur

        cur = lax.fori_loop(1, nch // 2, pair, cur)

        wait_gathers(1)
        fire_copyout(nch - 1, 1)
        wait_copyout(0)
        wait_copyout(1)

        def fire_block(b, carry):
            o = pl.multiple_of(b * SUB, SUB)
            pltpu.async_copy(unk_hbm.at[uid_v.at[pl.ds(o, SUB)]],
                             ubuf_v, sem_u).wait()
            pltpu.async_copy(ubuf_v, out_hbm.at[upos_v.at[pl.ds(o, SUB)]],
                             sem_u).wait()
            return carry

        nfull = lax.shift_right_logical(cur, 7)
        lax.fori_loop(0, nfull, fire_block, 0)

        rem = lax.bitwise_and(cur, SUB - 1)

        @pl.when(rem > 0)
        def _flush():
            last = jnp.full((16,), cur - 1, jnp.int32)
            padid = plsc.load_gather(uid_v, [last])
            padpos = plsc.load_gather(upos_v, [last])
            for k in range(SUB // 16):
                offs = jnp.full((16,), k * 16, jnp.int32) + lane \
                    + jnp.full((16,), lax.shift_left(nfull, 7), jnp.int32)
                mm = offs >= jnp.full((16,), cur, jnp.int32)
                plsc.store_scatter(uid_v, [offs], padid, mask=mm)
                plsc.store_scatter(upos_v, [offs], padpos, mask=mm)
            fire_block(nfull, 0)

    return body


def kernel(context, glove_table, unk_table):
    b, t = context.shape
    L = b * t
    NW = 32
    per_w = L // NW
    assert per_w % C == 0
    flat = context.reshape(L)
    out = _make_kernel(L, NW, per_w)(flat, glove_table, unk_table)
    return out.reshape(b, t, D)
